# baseline (speedup 1.0000x reference)
import dataclasses
import functools

import jax
import jax.numpy as jnp
from jax import lax
from jax.experimental import pallas as pl
from jax.experimental.pallas import tpu as pltpu
from jax.experimental.pallas import tpu_sc as plsc

_T = 128
_NCHUNK = 4
_D = 64
_R = 2048
_CPAD = _T * 16 * 6


def _build_plan(row, col, n, n_pad, ch, e_pad):
    deg = jnp.ones((n,), jnp.float32).at[col].add(1.0)
    b = col // ch
    onehot = (b[:, None] == jnp.arange(_NCHUNK, dtype=jnp.int32)[None, :])
    ranks = jnp.cumsum(onehot.astype(jnp.int32), axis=0)
    counts = ranks[-1]
    padlen = ((counts + _CPAD - 1) // _CPAD) * _CPAD
    pstart = jnp.concatenate(
        [jnp.zeros((1,), jnp.int32), jnp.cumsum(padlen)]).astype(jnp.int32)
    rank_i = jnp.take_along_axis(ranks, b[:, None], axis=1)[:, 0] - 1
    pos = pstart[b] + rank_i
    p = jnp.arange(e_pad, dtype=jnp.int32)
    row_pad = jnp.zeros((e_pad,), jnp.int32).at[pos].set(
        row, unique_indices=True)
    col_pad = (ch + (p & 15)).at[pos].set(col - b * ch, unique_indices=True)
    idx = jnp.stack([row_pad.reshape(-1, _T), col_pad.reshape(-1, _T)],
                    axis=1)
    bounds = jnp.zeros((16,), jnp.int32).at[:_NCHUNK + 1].set(pstart // _T)
    return idx, bounds, deg


def _make_prop(n_pad, e_pad, ch):
    acc_rows = ch + 16
    rows_w = ch // 16
    mesh = plsc.VectorSubcoreMesh(core_axis_name="c", subcore_axis_name="s")
    cp = pltpu.CompilerParams()
    if "needs_layout_passes" in pltpu.CompilerParams.__dataclass_fields__:
        cp = dataclasses.replace(cp, needs_layout_passes=False)
    if "use_tc_tiling_on_sc" in pltpu.CompilerParams.__dataclass_fields__:
        cp = dataclasses.replace(cp, use_tc_tiling_on_sc=False)

    @functools.partial(
        pl.kernel,
        out_type=jax.ShapeDtypeStruct((n_pad, _D), jnp.float32),
        mesh=mesh,
        compiler_params=cp,
        scratch_types=[
            pltpu.VMEM((6, 2, _T), jnp.int32),
            pltpu.VMEM((3, _T, _D), jnp.float32),
            pltpu.VMEM((16,), jnp.int32),
            pltpu.VMEM_SHARED((acc_rows, _D), jnp.float32),
            pltpu.SemaphoreType.DMA,
            pltpu.SemaphoreType.DMA,
            pltpu.SemaphoreType.DMA,
            pltpu.SemaphoreType.DMA,
            pltpu.SemaphoreType.DMA,
            pltpu.SemaphoreType.DMA,
            pltpu.SemaphoreType.DMA,
            pltpu.SemaphoreType.DMA,
            pltpu.SemaphoreType.DMA,
        ],
    )
    def prop(v_hbm, idx_hbm, bounds_hbm, y_hbm,
             idxv, rows, bnd, acc,
             i0, i1, i2, i3, i4, i5, g0, g1, g2):
        core = lax.axis_index("c")
        sid = lax.axis_index("s")
        isems = (i0, i1, i2, i3, i4, i5)
        gsems = (g0, g1, g2)
        pltpu.sync_copy(bounds_hbm, bnd)
        bvec = bnd[...]
        lanes = lax.broadcasted_iota(jnp.int32, (16,), 0)

        def extract(k):
            return jnp.sum(jnp.where(lanes == k, bvec, 0))

        def issue_i(t, q):
            pltpu.async_copy(idx_hbm.at[t], idxv.at[q], isems[q])

        def wait_i(q):
            pltpu.make_async_copy(idx_hbm.at[0], idxv.at[q], isems[q]).wait()

        def issue_g(q, b):
            pltpu.async_copy(v_hbm.at[idxv.at[q, 0]], rows.at[b], gsems[b])

        def wait_g(b):
            pltpu.make_async_copy(
                v_hbm.at[idxv.at[0, 0]], rows.at[b], gsems[b]).wait()

        for cc in range(_NCHUNK // 2):
            chunk = core * (_NCHUNK // 2) + cc
            ts = extract(chunk)
            te = extract(chunk + 1)
            node_base = chunk * ch
            pltpu.sync_copy(
                v_hbm.at[pl.ds(node_base + sid * rows_w, rows_w)],
                acc.at[pl.ds(sid * rows_w, rows_w)])
            plsc.subcore_barrier()
            npers = (te - ts) // 16
            base = ts + sid * npers

            for q in range(6):
                @pl.when(q < npers)
                def _(q=q):
                    issue_i(base + q, q)
            for b in range(3):
                @pl.when(b < npers)
                def _(b=b):
                    wait_i(b)
                    issue_g(b, b)

            def body(g, carry):
                for u in range(6):
                    j = g * 6 + u
                    b = u % 3
                    q = u
                    wait_g(b)
                    pltpu.sync_copy(rows.at[b], acc.at[idxv.at[q, 1]],
                                    add=True)

                    @pl.when(j + 6 < npers)
                    def _(j=j, q=q):
                        issue_i(base + j + 6, q)

                    @pl.when(j + 3 < npers)
                    def _(u=u, b=b):
                        wait_i((u + 3) % 6)
                        issue_g((u + 3) % 6, b)
                return carry

            lax.fori_loop(0, npers // 6, body, 0)
            plsc.subcore_barrier()
            pltpu.sync_copy(
                acc.at[pl.ds(sid * rows_w, rows_w)],
                y_hbm.at[pl.ds(node_base + sid * rows_w, rows_w)])
            plsc.subcore_barrier()

    return prop


def _colsum(x, n_valid):
    n_pad, d = x.shape
    nb = n_pad // _R

    def body(x_ref, o_ref):
        i = pl.program_id(0)
        rowid = i * _R + lax.broadcasted_iota(jnp.int32, (_R, 1), 0)
        s = jnp.sum(jnp.where(rowid < n_valid, x_ref[...], 0.0), axis=0,
                    keepdims=True)

        @pl.when(i == 0)
        def _():
            o_ref[...] = s

        @pl.when(i > 0)
        def _():
            o_ref[...] += s

    return pl.pallas_call(
        body,
        grid=(nb,),
        in_specs=[pl.BlockSpec((_R, d), lambda i: (i, 0))],
        out_specs=pl.BlockSpec((1, d), lambda i: (0, 0)),
        out_shape=jax.ShapeDtypeStruct((1, d), jnp.float32),
    )(x)


def _dense(x, w, a, n_valid):
    n_pad, din = x.shape
    dout = w.shape[1]
    nb = n_pad // _R
    s = _colsum(x, n_valid)
    inv_n = 1.0 / float(n_valid)

    def body(x_ref, s_ref, w_ref, a_ref, o_ref):
        xc = x_ref[...] - s_ref[...] * inv_n
        z = jnp.dot(xc, w_ref[...], preferred_element_type=jnp.float32)
        o_ref[...] = jnp.where(z >= 0, z, a_ref[...] * z)

    return pl.pallas_call(
        body,
        grid=(nb,),
        in_specs=[
            pl.BlockSpec((_R, din), lambda i: (i, 0)),
            pl.BlockSpec((1, din), lambda i: (0, 0)),
            pl.BlockSpec((din, dout), lambda i: (0, 0)),
            pl.BlockSpec((1, dout), lambda i: (0, 0)),
        ],
        out_specs=pl.BlockSpec((_R, dout), lambda i: (i, 0)),
        out_shape=jax.ShapeDtypeStruct((n_pad, dout), jnp.float32),
    )(x, s, w, a)


def _ssgc_start(xa, xb, c0, c1, dinv, alpha):
    n_pad, d = xa.shape
    nb = n_pad // _R

    def body(a_ref, b_ref, c0_ref, c1_ref, di_ref, v_ref, g_ref):
        mix = c0_ref[0, 0] * a_ref[...] + c1_ref[0, 0] * b_ref[...]
        v = mix * di_ref[...]
        v_ref[...] = v
        g_ref[...] = alpha * v

    return pl.pallas_call(
        body,
        grid=(nb,),
        in_specs=[
            pl.BlockSpec((_R, d), lambda i: (i, 0)),
            pl.BlockSpec((_R, d), lambda i: (i, 0)),
            pl.BlockSpec((1, 1), lambda i: (0, 0)),
            pl.BlockSpec((1, 1), lambda i: (0, 0)),
            pl.BlockSpec((_R, 1), lambda i: (i, 0)),
        ],
        out_specs=[
            pl.BlockSpec((_R, d), lambda i: (i, 0)),
            pl.BlockSpec((_R, d), lambda i: (i, 0)),
        ],
        out_shape=[
            jax.ShapeDtypeStruct((n_pad, d), jnp.float32),
            jax.ShapeDtypeStruct((n_pad, d), jnp.float32),
        ],
    )(xa, xb, c0, c1, dinv)


def _ssgc_step(y, g, dinv2, coef):
    n_pad, d = y.shape
    nb = n_pad // _R

    def body(y_ref, g_ref, d_ref, v_ref, go_ref):
        v = y_ref[...] * d_ref[...]
        v_ref[...] = v
        go_ref[...] = g_ref[...] + coef * v

    return pl.pallas_call(
        body,
        grid=(nb,),
        in_specs=[
            pl.BlockSpec((_R, d), lambda i: (i, 0)),
            pl.BlockSpec((_R, d), lambda i: (i, 0)),
            pl.BlockSpec((_R, 1), lambda i: (i, 0)),
        ],
        out_specs=[
            pl.BlockSpec((_R, d), lambda i: (i, 0)),
            pl.BlockSpec((_R, d), lambda i: (i, 0)),
        ],
        out_shape=[
            jax.ShapeDtypeStruct((n_pad, d), jnp.float32),
            jax.ShapeDtypeStruct((n_pad, d), jnp.float32),
        ],
    )(y, g, dinv2)


def _ssgc_final(y, g, dinv2, sq, coef):
    n_pad, d = y.shape
    nb = n_pad // _R

    def body(y_ref, g_ref, d_ref, s_ref, h_ref):
        v = y_ref[...] * d_ref[...]
        h_ref[...] = (g_ref[...] + coef * v) * s_ref[...]

    return pl.pallas_call(
        body,
        grid=(nb,),
        in_specs=[
            pl.BlockSpec((_R, d), lambda i: (i, 0)),
            pl.BlockSpec((_R, d), lambda i: (i, 0)),
            pl.BlockSpec((_R, 1), lambda i: (i, 0)),
            pl.BlockSpec((_R, 1), lambda i: (i, 0)),
        ],
        out_specs=pl.BlockSpec((_R, d), lambda i: (i, 0)),
        out_shape=jax.ShapeDtypeStruct((n_pad, d), jnp.float32),
    )(y, g, dinv2, sq)


def _mix3(xa, xb, xc, w0, w1, w2):
    n_pad, d = xa.shape
    nb = n_pad // _R

    def body(a_ref, b_ref, c_ref, w0_ref, w1_ref, w2_ref, o_ref):
        o_ref[...] = (w0_ref[0, 0] * a_ref[...] + w1_ref[0, 0] * b_ref[...]
                      + w2_ref[0, 0] * c_ref[...])

    return pl.pallas_call(
        body,
        grid=(nb,),
        in_specs=[
            pl.BlockSpec((_R, d), lambda i: (i, 0)),
            pl.BlockSpec((_R, d), lambda i: (i, 0)),
            pl.BlockSpec((_R, d), lambda i: (i, 0)),
            pl.BlockSpec((1, 1), lambda i: (0, 0)),
            pl.BlockSpec((1, 1), lambda i: (0, 0)),
            pl.BlockSpec((1, 1), lambda i: (0, 0)),
        ],
        out_specs=pl.BlockSpec((_R, d), lambda i: (i, 0)),
        out_shape=jax.ShapeDtypeStruct((n_pad, d), jnp.float32),
    )(xa, xb, xc, w0, w1, w2)


def _final(xa, xb, c0, c1, w, b):
    n_pad, din = xa.shape
    dout = w.shape[1]
    nb = n_pad // _R

    def body(a_ref, b2_ref, c0_ref, c1_ref, w_ref, bias_ref, o_ref):
        mix = c0_ref[0, 0] * a_ref[...] + c1_ref[0, 0] * b2_ref[...]
        o_ref[...] = jnp.dot(mix, w_ref[...],
                             preferred_element_type=jnp.float32) + bias_ref[...]

    return pl.pallas_call(
        body,
        grid=(nb,),
        in_specs=[
            pl.BlockSpec((_R, din), lambda i: (i, 0)),
            pl.BlockSpec((_R, din), lambda i: (i, 0)),
            pl.BlockSpec((1, 1), lambda i: (0, 0)),
            pl.BlockSpec((1, 1), lambda i: (0, 0)),
            pl.BlockSpec((din, dout), lambda i: (0, 0)),
            pl.BlockSpec((1, dout), lambda i: (0, 0)),
        ],
        out_specs=pl.BlockSpec((_R, dout), lambda i: (i, 0)),
        out_shape=jax.ShapeDtypeStruct((n_pad, dout), jnp.float32),
    )(xa, xb, c0, c1, w, b)


def kernel(x, edges, W1, b1, W2, b2, W3, b3, W4, b4, W5, b5, W6, b6,
           W7, b7, W8, b8, a1, a2, a3, a4, a5, a6, a7,
           p0, p1, p2, p3, p4):
    n = x.shape[0]
    e = edges.shape[1]
    ch = ((n + _NCHUNK - 1) // _NCHUNK + _T - 1) // _T * _T
    n_pad = ch * _NCHUNK
    e_pad = e + _NCHUNK * _CPAD
    alpha = 0.05

    row = edges[0].astype(jnp.int32)
    col = edges[1].astype(jnp.int32)
    idx, bounds, deg = _build_plan(row, col, n, n_pad, ch, e_pad)

    dinv = jnp.zeros((n_pad, 1), jnp.float32).at[:n, 0].set(deg ** -0.5)
    dinv2 = jnp.zeros((n_pad, 1), jnp.float32).at[:n, 0].set(1.0 / deg)
    sq = jnp.zeros((n_pad, 1), jnp.float32).at[:n, 0].set(jnp.sqrt(deg))

    prop = _make_prop(n_pad, e_pad, ch)

    def ssgc_block(xa, xb, c0, c1, k_steps, w, a):
        coef = (1.0 - alpha) / k_steps
        v, g = _ssgc_start(xa, xb, c0, c1, dinv, alpha)
        for k in range(k_steps):
            y = prop(v, idx, bounds)
            if k < k_steps - 1:
                v, g = _ssgc_step(y, g, dinv2, coef)
            else:
                hx = _ssgc_final(y, g, dinv2, sq, coef)
        return _dense(hx, w, a, n)

    def s11(v):
        return jnp.reshape(v.astype(jnp.float32), (1, 1))

    one = jnp.ones((1, 1), jnp.float32)
    zero = jnp.zeros((1, 1), jnp.float32)

    x_pad = jnp.zeros((n_pad, 8), jnp.float32).at[:n, :6].set(x)
    w1p = jnp.zeros((8, 32), jnp.float32).at[:6].set(W1)

    x1 = _dense(x_pad, w1p, a1.reshape(1, -1), n)
    x2 = _dense(x1, W2, a2.reshape(1, -1), n)

    x3 = ssgc_block(x2, x2, one, zero, 3, W3, a3.reshape(1, -1))
    x4 = ssgc_block(x2, x3, s11(1.0 - p0), s11(p0), 4, W4, a4.reshape(1, -1))
    x5 = ssgc_block(x3, x4, s11(1.0 - p1), s11(p1), 4, W5, a5.reshape(1, -1))
    x6 = ssgc_block(x4, x5, s11(1.0 - p2), s11(p2), 3, W6, a6.reshape(1, -1))

    wts = jax.nn.softmax(p3)
    res4 = _mix3(x2, x5, x6, s11(wts[0]), s11(wts[1]), s11(wts[2]))
    x7 = _dense(res4, W7, a7.reshape(1, -1), n)

    out = _final(x1, x7, s11(1.0 - p4), s11(p4), W8, b8.reshape(1, -1))
    return out[:n]


# speedup vs baseline: 1.1054x; 1.1054x over previous
import dataclasses
import functools

import jax
import jax.numpy as jnp
from jax import lax
from jax.experimental import pallas as pl
from jax.experimental.pallas import tpu as pltpu
from jax.experimental.pallas import tpu_sc as plsc

_T = 128
_NCHUNK = 4
_D = 64
_R = 2048
_CPAD = _T * 16 * 4


def _build_plan(row, col, n, n_pad, ch, e_pad):
    b = col // ch
    onehot = (b[:, None] == jnp.arange(_NCHUNK, dtype=jnp.int32)[None, :])
    ranks = jnp.cumsum(onehot.astype(jnp.int32), axis=0)
    counts = ranks[-1]
    padlen = ((counts + _CPAD - 1) // _CPAD) * _CPAD
    pstart = jnp.concatenate(
        [jnp.zeros((1,), jnp.int32), jnp.cumsum(padlen)]).astype(jnp.int32)
    rank_i = jnp.take_along_axis(ranks, b[:, None], axis=1)[:, 0] - 1
    pos = pstart[b] + rank_i
    p = jnp.arange(e_pad, dtype=jnp.int32)
    row_pad = jnp.zeros((e_pad,), jnp.int32).at[pos].set(
        row, unique_indices=True)
    col_pad = (ch + (p & 15)).at[pos].set(col - b * ch, unique_indices=True)
    idx = jnp.stack([row_pad.reshape(-1, _T), col_pad.reshape(-1, _T)],
                    axis=1)
    bounds = jnp.zeros((16,), jnp.int32).at[:_NCHUNK + 1].set(pstart // _T)
    return idx, bounds


def _make_prop(n_pad, e_pad, ch):
    acc_rows = ch + 16
    rows_w = ch // 16
    mesh = plsc.VectorSubcoreMesh(core_axis_name="c", subcore_axis_name="s")
    cp = pltpu.CompilerParams()
    if "needs_layout_passes" in pltpu.CompilerParams.__dataclass_fields__:
        cp = dataclasses.replace(cp, needs_layout_passes=False)
    if "use_tc_tiling_on_sc" in pltpu.CompilerParams.__dataclass_fields__:
        cp = dataclasses.replace(cp, use_tc_tiling_on_sc=False)

    @functools.partial(
        pl.kernel,
        out_type=jax.ShapeDtypeStruct((n_pad, _D), jnp.float32),
        mesh=mesh,
        compiler_params=cp,
        scratch_types=[
            pltpu.VMEM((4, 2, _T), jnp.int32),
            pltpu.VMEM((2, _T, _D), jnp.float32),
            pltpu.VMEM((16,), jnp.int32),
            pltpu.VMEM_SHARED((acc_rows, _D), jnp.float32),
            pltpu.SemaphoreType.DMA,
            pltpu.SemaphoreType.DMA,
            pltpu.SemaphoreType.DMA,
            pltpu.SemaphoreType.DMA,
            pltpu.SemaphoreType.DMA,
            pltpu.SemaphoreType.DMA,
        ],
    )
    def prop(v_hbm, idx_hbm, bounds_hbm, y_hbm,
             idxv, rows, bnd, acc, i0, i1, i2, i3, g0, g1):
        core = lax.axis_index("c")
        sid = lax.axis_index("s")
        isems = (i0, i1, i2, i3)
        gsems = (g0, g1)
        pltpu.sync_copy(bounds_hbm, bnd)
        bvec = bnd[...]
        lanes = lax.broadcasted_iota(jnp.int32, (16,), 0)

        def extract(k):
            return jnp.sum(jnp.where(lanes == k, bvec, 0))

        def issue_i(t, q):
            pltpu.async_copy(idx_hbm.at[t], idxv.at[q], isems[q])

        def wait_i(q):
            pltpu.make_async_copy(idx_hbm.at[0], idxv.at[q], isems[q]).wait()

        def issue_g(q, b):
            pltpu.async_copy(v_hbm.at[idxv.at[q, 0]], rows.at[b], gsems[b])

        def wait_g(b):
            pltpu.make_async_copy(
                v_hbm.at[idxv.at[0, 0]], rows.at[b], gsems[b]).wait()

        for cc in range(_NCHUNK // 2):
            chunk = core * (_NCHUNK // 2) + cc
            ts = extract(chunk)
            te = extract(chunk + 1)
            node_base = chunk * ch
            pltpu.sync_copy(
                v_hbm.at[pl.ds(node_base + sid * rows_w, rows_w)],
                acc.at[pl.ds(sid * rows_w, rows_w)])
            plsc.subcore_barrier()
            npers = (te - ts) // 16
            base = ts + sid * npers

            for q in range(4):
                @pl.when(q < npers)
                def _(q=q):
                    issue_i(base + q, q)
            for b in range(2):
                @pl.when(b < npers)
                def _(b=b):
                    wait_i(b)
                    issue_g(b, b)

            def body(g, carry):
                for u in range(4):
                    j = g * 4 + u
                    b = u % 2
                    q = u
                    wait_g(b)
                    pltpu.sync_copy(rows.at[b], acc.at[idxv.at[q, 1]],
                                    add=True)

                    @pl.when(j + 4 < npers)
                    def _(j=j, q=q):
                        issue_i(base + j + 4, q)

                    @pl.when(j + 2 < npers)
                    def _(u=u, b=b):
                        wait_i((u + 2) % 4)
                        issue_g((u + 2) % 4, b)
                return carry

            lax.fori_loop(0, npers // 4, body, 0)
            plsc.subcore_barrier()
            pltpu.sync_copy(
                acc.at[pl.ds(sid * rows_w, rows_w)],
                y_hbm.at[pl.ds(node_base + sid * rows_w, rows_w)])
            plsc.subcore_barrier()

    return prop


def _colsum(x, n_valid):
    n_pad, d = x.shape
    nb = n_pad // _R

    def body(x_ref, o_ref):
        i = pl.program_id(0)
        rowid = i * _R + lax.broadcasted_iota(jnp.int32, (_R, 1), 0)
        s = jnp.sum(jnp.where(rowid < n_valid, x_ref[...], 0.0), axis=0,
                    keepdims=True)

        @pl.when(i == 0)
        def _():
            o_ref[...] = s

        @pl.when(i > 0)
        def _():
            o_ref[...] += s

    return pl.pallas_call(
        body,
        grid=(nb,),
        in_specs=[pl.BlockSpec((_R, d), lambda i: (i, 0))],
        out_specs=pl.BlockSpec((1, d), lambda i: (0, 0)),
        out_shape=jax.ShapeDtypeStruct((1, d), jnp.float32),
    )(x)


def _dense(x, w, a, n_valid):
    n_pad, din = x.shape
    dout = w.shape[1]
    nb = n_pad // _R
    s = _colsum(x, n_valid)
    inv_n = 1.0 / float(n_valid)

    def body(x_ref, s_ref, w_ref, a_ref, o_ref):
        xc = x_ref[...] - s_ref[...] * inv_n
        z = jnp.dot(xc, w_ref[...], preferred_element_type=jnp.float32)
        o_ref[...] = jnp.where(z >= 0, z, a_ref[...] * z)

    return pl.pallas_call(
        body,
        grid=(nb,),
        in_specs=[
            pl.BlockSpec((_R, din), lambda i: (i, 0)),
            pl.BlockSpec((1, din), lambda i: (0, 0)),
            pl.BlockSpec((din, dout), lambda i: (0, 0)),
            pl.BlockSpec((1, dout), lambda i: (0, 0)),
        ],
        out_specs=pl.BlockSpec((_R, dout), lambda i: (i, 0)),
        out_shape=jax.ShapeDtypeStruct((n_pad, dout), jnp.float32),
    )(x, s, w, a)


def _ssgc_start(xa, xb, c0, c1, dinv, alpha):
    n_pad, d = xa.shape
    nb = n_pad // _R

    def body(a_ref, b_ref, c0_ref, c1_ref, di_ref, v_ref, g_ref):
        mix = c0_ref[0, 0] * a_ref[...] + c1_ref[0, 0] * b_ref[...]
        v = mix * di_ref[...]
        v_ref[...] = v
        g_ref[...] = alpha * v

    return pl.pallas_call(
        body,
        grid=(nb,),
        in_specs=[
            pl.BlockSpec((_R, d), lambda i: (i, 0)),
            pl.BlockSpec((_R, d), lambda i: (i, 0)),
            pl.BlockSpec((1, 1), lambda i: (0, 0)),
            pl.BlockSpec((1, 1), lambda i: (0, 0)),
            pl.BlockSpec((_R, 1), lambda i: (i, 0)),
        ],
        out_specs=[
            pl.BlockSpec((_R, d), lambda i: (i, 0)),
            pl.BlockSpec((_R, d), lambda i: (i, 0)),
        ],
        out_shape=[
            jax.ShapeDtypeStruct((n_pad, d), jnp.float32),
            jax.ShapeDtypeStruct((n_pad, d), jnp.float32),
        ],
    )(xa, xb, c0, c1, dinv)


def _ssgc_step(y, g, dinv2, coef):
    n_pad, d = y.shape
    nb = n_pad // _R

    def body(y_ref, g_ref, d_ref, v_ref, go_ref):
        v = y_ref[...] * d_ref[...]
        v_ref[...] = v
        go_ref[...] = g_ref[...] + coef * v

    return pl.pallas_call(
        body,
        grid=(nb,),
        in_specs=[
            pl.BlockSpec((_R, d), lambda i: (i, 0)),
            pl.BlockSpec((_R, d), lambda i: (i, 0)),
            pl.BlockSpec((_R, 1), lambda i: (i, 0)),
        ],
        out_specs=[
            pl.BlockSpec((_R, d), lambda i: (i, 0)),
            pl.BlockSpec((_R, d), lambda i: (i, 0)),
        ],
        out_shape=[
            jax.ShapeDtypeStruct((n_pad, d), jnp.float32),
            jax.ShapeDtypeStruct((n_pad, d), jnp.float32),
        ],
    )(y, g, dinv2)


def _ssgc_final(y, g, dinv2, sq, coef):
    n_pad, d = y.shape
    nb = n_pad // _R

    def body(y_ref, g_ref, d_ref, s_ref, h_ref):
        v = y_ref[...] * d_ref[...]
        h_ref[...] = (g_ref[...] + coef * v) * s_ref[...]

    return pl.pallas_call(
        body,
        grid=(nb,),
        in_specs=[
            pl.BlockSpec((_R, d), lambda i: (i, 0)),
            pl.BlockSpec((_R, d), lambda i: (i, 0)),
            pl.BlockSpec((_R, 1), lambda i: (i, 0)),
            pl.BlockSpec((_R, 1), lambda i: (i, 0)),
        ],
        out_specs=pl.BlockSpec((_R, d), lambda i: (i, 0)),
        out_shape=jax.ShapeDtypeStruct((n_pad, d), jnp.float32),
    )(y, g, dinv2, sq)


def _mix3(xa, xb, xc, w0, w1, w2):
    n_pad, d = xa.shape
    nb = n_pad // _R

    def body(a_ref, b_ref, c_ref, w0_ref, w1_ref, w2_ref, o_ref):
        o_ref[...] = (w0_ref[0, 0] * a_ref[...] + w1_ref[0, 0] * b_ref[...]
                      + w2_ref[0, 0] * c_ref[...])

    return pl.pallas_call(
        body,
        grid=(nb,),
        in_specs=[
            pl.BlockSpec((_R, d), lambda i: (i, 0)),
            pl.BlockSpec((_R, d), lambda i: (i, 0)),
            pl.BlockSpec((_R, d), lambda i: (i, 0)),
            pl.BlockSpec((1, 1), lambda i: (0, 0)),
            pl.BlockSpec((1, 1), lambda i: (0, 0)),
            pl.BlockSpec((1, 1), lambda i: (0, 0)),
        ],
        out_specs=pl.BlockSpec((_R, d), lambda i: (i, 0)),
        out_shape=jax.ShapeDtypeStruct((n_pad, d), jnp.float32),
    )(xa, xb, xc, w0, w1, w2)


def _final(xa, xb, c0, c1, w, b):
    n_pad, din = xa.shape
    dout = w.shape[1]
    nb = n_pad // _R

    def body(a_ref, b2_ref, c0_ref, c1_ref, w_ref, bias_ref, o_ref):
        mix = c0_ref[0, 0] * a_ref[...] + c1_ref[0, 0] * b2_ref[...]
        o_ref[...] = jnp.dot(mix, w_ref[...],
                             preferred_element_type=jnp.float32) + bias_ref[...]

    return pl.pallas_call(
        body,
        grid=(nb,),
        in_specs=[
            pl.BlockSpec((_R, din), lambda i: (i, 0)),
            pl.BlockSpec((_R, din), lambda i: (i, 0)),
            pl.BlockSpec((1, 1), lambda i: (0, 0)),
            pl.BlockSpec((1, 1), lambda i: (0, 0)),
            pl.BlockSpec((din, dout), lambda i: (0, 0)),
            pl.BlockSpec((1, dout), lambda i: (0, 0)),
        ],
        out_specs=pl.BlockSpec((_R, dout), lambda i: (i, 0)),
        out_shape=jax.ShapeDtypeStruct((n_pad, dout), jnp.float32),
    )(xa, xb, c0, c1, w, b)


def kernel(x, edges, W1, b1, W2, b2, W3, b3, W4, b4, W5, b5, W6, b6,
           W7, b7, W8, b8, a1, a2, a3, a4, a5, a6, a7,
           p0, p1, p2, p3, p4):
    n = x.shape[0]
    e = edges.shape[1]
    ch = ((n + _NCHUNK - 1) // _NCHUNK + _T - 1) // _T * _T
    n_pad = ch * _NCHUNK
    e_pad = e + _NCHUNK * _CPAD
    alpha = 0.05

    row = edges[0].astype(jnp.int32)
    col = edges[1].astype(jnp.int32)
    idx, bounds = _build_plan(row, col, n, n_pad, ch, e_pad)

    prop = _make_prop(n_pad, e_pad, ch)

    deg = prop(jnp.ones((n_pad, _D), jnp.float32), idx, bounds)[:n, 0]

    dinv = jnp.zeros((n_pad, 1), jnp.float32).at[:n, 0].set(deg ** -0.5)
    dinv2 = jnp.zeros((n_pad, 1), jnp.float32).at[:n, 0].set(1.0 / deg)
    sq = jnp.zeros((n_pad, 1), jnp.float32).at[:n, 0].set(jnp.sqrt(deg))

    def ssgc_block(xa, xb, c0, c1, k_steps, w, a):
        coef = (1.0 - alpha) / k_steps
        v, g = _ssgc_start(xa, xb, c0, c1, dinv, alpha)
        for k in range(k_steps):
            y = prop(v, idx, bounds)
            if k < k_steps - 1:
                v, g = _ssgc_step(y, g, dinv2, coef)
            else:
                hx = _ssgc_final(y, g, dinv2, sq, coef)
        return _dense(hx, w, a, n)

    def s11(v):
        return jnp.reshape(v.astype(jnp.float32), (1, 1))

    one = jnp.ones((1, 1), jnp.float32)
    zero = jnp.zeros((1, 1), jnp.float32)

    x_pad = jnp.zeros((n_pad, 8), jnp.float32).at[:n, :6].set(x)
    w1p = jnp.zeros((8, 32), jnp.float32).at[:6].set(W1)

    x1 = _dense(x_pad, w1p, a1.reshape(1, -1), n)
    x2 = _dense(x1, W2, a2.reshape(1, -1), n)

    x3 = ssgc_block(x2, x2, one, zero, 3, W3, a3.reshape(1, -1))
    x4 = ssgc_block(x2, x3, s11(1.0 - p0), s11(p0), 4, W4, a4.reshape(1, -1))
    x5 = ssgc_block(x3, x4, s11(1.0 - p1), s11(p1), 4, W5, a5.reshape(1, -1))
    x6 = ssgc_block(x4, x5, s11(1.0 - p2), s11(p2), 3, W6, a6.reshape(1, -1))

    wts = jax.nn.softmax(p3)
    res4 = _mix3(x2, x5, x6, s11(wts[0]), s11(wts[1]), s11(wts[2]))
    x7 = _dense(res4, W7, a7.reshape(1, -1), n)

    out = _final(x1, x7, s11(1.0 - p4), s11(p4), W8, b8.reshape(1, -1))
    return out[:n]


# speedup vs baseline: 1.1701x; 1.0585x over previous
import dataclasses
import functools

import jax
import jax.numpy as jnp
from jax import lax
from jax.experimental import pallas as pl
from jax.experimental.pallas import tpu as pltpu
from jax.experimental.pallas import tpu_sc as plsc

_T = 128
_NCHUNK = 4
_D = 64
_R = 2048
_CPAD = _T * 16 * 4


def _build_plan(row, col, n, n_pad, ch, e_pad):
    b = col // ch
    onehot = (b[:, None] == jnp.arange(_NCHUNK, dtype=jnp.int32)[None, :])
    ranks = jnp.cumsum(onehot.astype(jnp.int32), axis=0)
    counts = ranks[-1]
    padlen = ((counts + _CPAD - 1) // _CPAD) * _CPAD
    pstart = jnp.concatenate(
        [jnp.zeros((1,), jnp.int32), jnp.cumsum(padlen)]).astype(jnp.int32)
    rank_i = jnp.sum(jnp.where(onehot, ranks, 0), axis=1) - 1
    pos = pstart[b] + rank_i
    p = jnp.arange(e_pad, dtype=jnp.int32)
    row_pad = jnp.zeros((e_pad,), jnp.int32).at[pos].set(
        row, unique_indices=True)
    col_pad = (ch + (p & 15)).at[pos].set(col - b * ch, unique_indices=True)
    idx = jnp.stack([row_pad.reshape(-1, _T), col_pad.reshape(-1, _T)],
                    axis=1)
    bounds = jnp.zeros((16,), jnp.int32).at[:_NCHUNK + 1].set(pstart // _T)
    return idx, bounds


def _make_prop(n_pad, e_pad, ch):
    acc_rows = ch + 16
    rows_w = ch // 16
    mesh = plsc.VectorSubcoreMesh(core_axis_name="c", subcore_axis_name="s")
    cp = pltpu.CompilerParams()
    if "needs_layout_passes" in pltpu.CompilerParams.__dataclass_fields__:
        cp = dataclasses.replace(cp, needs_layout_passes=False)
    if "use_tc_tiling_on_sc" in pltpu.CompilerParams.__dataclass_fields__:
        cp = dataclasses.replace(cp, use_tc_tiling_on_sc=False)

    @functools.partial(
        pl.kernel,
        out_type=jax.ShapeDtypeStruct((n_pad, _D), jnp.float32),
        mesh=mesh,
        compiler_params=cp,
        scratch_types=[
            pltpu.VMEM((4, 2, _T), jnp.int32),
            pltpu.VMEM((2, _T, _D), jnp.float32),
            pltpu.VMEM((16,), jnp.int32),
            pltpu.VMEM_SHARED((acc_rows, _D), jnp.float32),
            pltpu.SemaphoreType.DMA,
            pltpu.SemaphoreType.DMA,
            pltpu.SemaphoreType.DMA,
            pltpu.SemaphoreType.DMA,
            pltpu.SemaphoreType.DMA,
            pltpu.SemaphoreType.DMA,
        ],
    )
    def prop(v_hbm, idx_hbm, bounds_hbm, y_hbm,
             idxv, rows, bnd, acc, i0, i1, i2, i3, g0, g1):
        core = lax.axis_index("c")
        sid = lax.axis_index("s")
        isems = (i0, i1, i2, i3)
        gsems = (g0, g1)
        pltpu.sync_copy(bounds_hbm, bnd)
        bvec = bnd[...]
        lanes = lax.broadcasted_iota(jnp.int32, (16,), 0)

        def extract(k):
            return jnp.sum(jnp.where(lanes == k, bvec, 0))

        def issue_i(t, q):
            pltpu.async_copy(idx_hbm.at[t], idxv.at[q], isems[q])

        def wait_i(q):
            pltpu.make_async_copy(idx_hbm.at[0], idxv.at[q], isems[q]).wait()

        def issue_g(q, b):
            pltpu.async_copy(v_hbm.at[idxv.at[q, 0]], rows.at[b], gsems[b])

        def wait_g(b):
            pltpu.make_async_copy(
                v_hbm.at[idxv.at[0, 0]], rows.at[b], gsems[b]).wait()

        for cc in range(_NCHUNK // 2):
            chunk = core * (_NCHUNK // 2) + cc
            ts = extract(chunk)
            te = extract(chunk + 1)
            node_base = chunk * ch
            pltpu.sync_copy(
                v_hbm.at[pl.ds(node_base + sid * rows_w, rows_w)],
                acc.at[pl.ds(sid * rows_w, rows_w)])
            plsc.subcore_barrier()
            npers = (te - ts) // 16
            base = ts + sid * npers

            for q in range(4):
                @pl.when(q < npers)
                def _(q=q):
                    issue_i(base + q, q)
            for b in range(2):
                @pl.when(b < npers)
                def _(b=b):
                    wait_i(b)
                    issue_g(b, b)

            def body(g, carry):
                for u in range(4):
                    j = g * 4 + u
                    b = u % 2
                    q = u
                    wait_g(b)
                    pltpu.sync_copy(rows.at[b], acc.at[idxv.at[q, 1]],
                                    add=True)

                    @pl.when(j + 4 < npers)
                    def _(j=j, q=q):
                        issue_i(base + j + 4, q)

                    @pl.when(j + 2 < npers)
                    def _(u=u, b=b):
                        wait_i((u + 2) % 4)
                        issue_g((u + 2) % 4, b)
                return carry

            lax.fori_loop(0, npers // 4, body, 0)
            plsc.subcore_barrier()
            pltpu.sync_copy(
                acc.at[pl.ds(sid * rows_w, rows_w)],
                y_hbm.at[pl.ds(node_base + sid * rows_w, rows_w)])
            plsc.subcore_barrier()

    return prop


def _colsum(x, n_valid):
    n_pad, d = x.shape
    nb = n_pad // _R

    def body(x_ref, o_ref):
        i = pl.program_id(0)
        rowid = i * _R + lax.broadcasted_iota(jnp.int32, (_R, 1), 0)
        s = jnp.sum(jnp.where(rowid < n_valid, x_ref[...], 0.0), axis=0,
                    keepdims=True)

        @pl.when(i == 0)
        def _():
            o_ref[...] = s

        @pl.when(i > 0)
        def _():
            o_ref[...] += s

    return pl.pallas_call(
        body,
        grid=(nb,),
        in_specs=[pl.BlockSpec((_R, d), lambda i: (i, 0))],
        out_specs=pl.BlockSpec((1, d), lambda i: (0, 0)),
        out_shape=jax.ShapeDtypeStruct((1, d), jnp.float32),
    )(x)


def _dense(x, w, a, n_valid):
    n_pad, din = x.shape
    dout = w.shape[1]
    nb = n_pad // _R
    s = _colsum(x, n_valid)
    inv_n = 1.0 / float(n_valid)

    def body(x_ref, s_ref, w_ref, a_ref, o_ref):
        xc = x_ref[...] - s_ref[...] * inv_n
        z = jnp.dot(xc, w_ref[...], preferred_element_type=jnp.float32)
        o_ref[...] = jnp.where(z >= 0, z, a_ref[...] * z)

    return pl.pallas_call(
        body,
        grid=(nb,),
        in_specs=[
            pl.BlockSpec((_R, din), lambda i: (i, 0)),
            pl.BlockSpec((1, din), lambda i: (0, 0)),
            pl.BlockSpec((din, dout), lambda i: (0, 0)),
            pl.BlockSpec((1, dout), lambda i: (0, 0)),
        ],
        out_specs=pl.BlockSpec((_R, dout), lambda i: (i, 0)),
        out_shape=jax.ShapeDtypeStruct((n_pad, dout), jnp.float32),
    )(x, s, w, a)


def _ssgc_start(xa, xb, c0, c1, dinv, alpha):
    n_pad, d = xa.shape
    nb = n_pad // _R

    def body(a_ref, b_ref, c0_ref, c1_ref, di_ref, v_ref, g_ref):
        mix = c0_ref[0, 0] * a_ref[...] + c1_ref[0, 0] * b_ref[...]
        v = mix * di_ref[...]
        v_ref[...] = v
        g_ref[...] = alpha * v

    return pl.pallas_call(
        body,
        grid=(nb,),
        in_specs=[
            pl.BlockSpec((_R, d), lambda i: (i, 0)),
            pl.BlockSpec((_R, d), lambda i: (i, 0)),
            pl.BlockSpec((1, 1), lambda i: (0, 0)),
            pl.BlockSpec((1, 1), lambda i: (0, 0)),
            pl.BlockSpec((_R, 1), lambda i: (i, 0)),
        ],
        out_specs=[
            pl.BlockSpec((_R, d), lambda i: (i, 0)),
            pl.BlockSpec((_R, d), lambda i: (i, 0)),
        ],
        out_shape=[
            jax.ShapeDtypeStruct((n_pad, d), jnp.float32),
            jax.ShapeDtypeStruct((n_pad, d), jnp.float32),
        ],
    )(xa, xb, c0, c1, dinv)


def _ssgc_step(y, g, dinv2, coef):
    n_pad, d = y.shape
    nb = n_pad // _R

    def body(y_ref, g_ref, d_ref, v_ref, go_ref):
        v = y_ref[...] * d_ref[...]
        v_ref[...] = v
        go_ref[...] = g_ref[...] + coef * v

    return pl.pallas_call(
        body,
        grid=(nb,),
        in_specs=[
            pl.BlockSpec((_R, d), lambda i: (i, 0)),
            pl.BlockSpec((_R, d), lambda i: (i, 0)),
            pl.BlockSpec((_R, 1), lambda i: (i, 0)),
        ],
        out_specs=[
            pl.BlockSpec((_R, d), lambda i: (i, 0)),
            pl.BlockSpec((_R, d), lambda i: (i, 0)),
        ],
        out_shape=[
            jax.ShapeDtypeStruct((n_pad, d), jnp.float32),
            jax.ShapeDtypeStruct((n_pad, d), jnp.float32),
        ],
    )(y, g, dinv2)


def _ssgc_final(y, g, dinv2, sq, coef):
    n_pad, d = y.shape
    nb = n_pad // _R

    def body(y_ref, g_ref, d_ref, s_ref, h_ref):
        v = y_ref[...] * d_ref[...]
        h_ref[...] = (g_ref[...] + coef * v) * s_ref[...]

    return pl.pallas_call(
        body,
        grid=(nb,),
        in_specs=[
            pl.BlockSpec((_R, d), lambda i: (i, 0)),
            pl.BlockSpec((_R, d), lambda i: (i, 0)),
            pl.BlockSpec((_R, 1), lambda i: (i, 0)),
            pl.BlockSpec((_R, 1), lambda i: (i, 0)),
        ],
        out_specs=pl.BlockSpec((_R, d), lambda i: (i, 0)),
        out_shape=jax.ShapeDtypeStruct((n_pad, d), jnp.float32),
    )(y, g, dinv2, sq)


def _mix3(xa, xb, xc, w0, w1, w2):
    n_pad, d = xa.shape
    nb = n_pad // _R

    def body(a_ref, b_ref, c_ref, w0_ref, w1_ref, w2_ref, o_ref):
        o_ref[...] = (w0_ref[0, 0] * a_ref[...] + w1_ref[0, 0] * b_ref[...]
                      + w2_ref[0, 0] * c_ref[...])

    return pl.pallas_call(
        body,
        grid=(nb,),
        in_specs=[
            pl.BlockSpec((_R, d), lambda i: (i, 0)),
            pl.BlockSpec((_R, d), lambda i: (i, 0)),
            pl.BlockSpec((_R, d), lambda i: (i, 0)),
            pl.BlockSpec((1, 1), lambda i: (0, 0)),
            pl.BlockSpec((1, 1), lambda i: (0, 0)),
            pl.BlockSpec((1, 1), lambda i: (0, 0)),
        ],
        out_specs=pl.BlockSpec((_R, d), lambda i: (i, 0)),
        out_shape=jax.ShapeDtypeStruct((n_pad, d), jnp.float32),
    )(xa, xb, xc, w0, w1, w2)


def _final(xa, xb, c0, c1, w, b):
    n_pad, din = xa.shape
    dout = w.shape[1]
    nb = n_pad // _R

    def body(a_ref, b2_ref, c0_ref, c1_ref, w_ref, bias_ref, o_ref):
        mix = c0_ref[0, 0] * a_ref[...] + c1_ref[0, 0] * b2_ref[...]
        o_ref[...] = jnp.dot(mix, w_ref[...],
                             preferred_element_type=jnp.float32) + bias_ref[...]

    return pl.pallas_call(
        body,
        grid=(nb,),
        in_specs=[
            pl.BlockSpec((_R, din), lambda i: (i, 0)),
            pl.BlockSpec((_R, din), lambda i: (i, 0)),
            pl.BlockSpec((1, 1), lambda i: (0, 0)),
            pl.BlockSpec((1, 1), lambda i: (0, 0)),
            pl.BlockSpec((din, dout), lambda i: (0, 0)),
            pl.BlockSpec((1, dout), lambda i: (0, 0)),
        ],
        out_specs=pl.BlockSpec((_R, dout), lambda i: (i, 0)),
        out_shape=jax.ShapeDtypeStruct((n_pad, dout), jnp.float32),
    )(xa, xb, c0, c1, w, b)


def kernel(x, edges, W1, b1, W2, b2, W3, b3, W4, b4, W5, b5, W6, b6,
           W7, b7, W8, b8, a1, a2, a3, a4, a5, a6, a7,
           p0, p1, p2, p3, p4):
    n = x.shape[0]
    e = edges.shape[1]
    ch = ((n + _NCHUNK - 1) // _NCHUNK + _T - 1) // _T * _T
    n_pad = ch * _NCHUNK
    e_pad = e + _NCHUNK * _CPAD
    alpha = 0.05

    row = edges[0].astype(jnp.int32)
    col = edges[1].astype(jnp.int32)
    idx, bounds = _build_plan(row, col, n, n_pad, ch, e_pad)

    prop = _make_prop(n_pad, e_pad, ch)

    deg = prop(jnp.ones((n_pad, _D), jnp.float32), idx, bounds)[:n, 0]

    dinv = jnp.zeros((n_pad, 1), jnp.float32).at[:n, 0].set(deg ** -0.5)
    dinv2 = jnp.zeros((n_pad, 1), jnp.float32).at[:n, 0].set(1.0 / deg)
    sq = jnp.zeros((n_pad, 1), jnp.float32).at[:n, 0].set(jnp.sqrt(deg))

    def ssgc_block(xa, xb, c0, c1, k_steps, w, a):
        coef = (1.0 - alpha) / k_steps
        v, g = _ssgc_start(xa, xb, c0, c1, dinv, alpha)
        for k in range(k_steps):
            y = prop(v, idx, bounds)
            if k < k_steps - 1:
                v, g = _ssgc_step(y, g, dinv2, coef)
            else:
                hx = _ssgc_final(y, g, dinv2, sq, coef)
        return _dense(hx, w, a, n)

    def s11(v):
        return jnp.reshape(v.astype(jnp.float32), (1, 1))

    one = jnp.ones((1, 1), jnp.float32)
    zero = jnp.zeros((1, 1), jnp.float32)

    x_pad = jnp.zeros((n_pad, 8), jnp.float32).at[:n, :6].set(x)
    w1p = jnp.zeros((8, 32), jnp.float32).at[:6].set(W1)

    x1 = _dense(x_pad, w1p, a1.reshape(1, -1), n)
    x2 = _dense(x1, W2, a2.reshape(1, -1), n)

    x3 = ssgc_block(x2, x2, one, zero, 3, W3, a3.reshape(1, -1))
    x4 = ssgc_block(x2, x3, s11(1.0 - p0), s11(p0), 4, W4, a4.reshape(1, -1))
    x5 = ssgc_block(x3, x4, s11(1.0 - p1), s11(p1), 4, W5, a5.reshape(1, -1))
    x6 = ssgc_block(x4, x5, s11(1.0 - p2), s11(p2), 3, W6, a6.reshape(1, -1))

    wts = jax.nn.softmax(p3)
    res4 = _mix3(x2, x5, x6, s11(wts[0]), s11(wts[1]), s11(wts[2]))
    x7 = _dense(res4, W7, a7.reshape(1, -1), n)

    out = _final(x1, x7, s11(1.0 - p4), s11(p4), W8, b8.reshape(1, -1))
    return out[:n]
